# baseline (speedup 1.0000x reference)
import functools
import math

import jax
import jax.numpy as jnp
from jax import lax
from jax.experimental import pallas as pl
from jax.experimental.pallas import tpu as pltpu
from jax.experimental.pallas import tpu_sc as plsc

D_MODEL = 64
SCALE = math.sqrt(D_MODEL)

NUM_CORES = 2
NUM_SUBCORES = 16
NUM_WORKERS = NUM_CORES * NUM_SUBCORES

CHUNK = 128


def _emb_body(x_hbm, table_hbm, out_hbm, idx_v, rows_v, sem, *, per_w, n_chunks):
    wid = lax.axis_index("s") * NUM_CORES + lax.axis_index("c")
    base = wid * per_w

    def chunk_body(ci, carry):
        off = base + ci * CHUNK
        pltpu.sync_copy(x_hbm.at[pl.ds(off, CHUNK)], idx_v)
        pltpu.async_copy(table_hbm.at[idx_v], rows_v, sem).wait()

        def row_body(j, c):
            for k in range(D_MODEL // 16):
                sl = pl.ds(k * 16, 16)
                rows_v[j, sl] = rows_v[j, sl] * SCALE
            return c

        lax.fori_loop(0, CHUNK, row_body, 0, unroll=2)
        pltpu.sync_copy(rows_v, out_hbm.at[pl.ds(off, CHUNK)])
        return carry

    lax.fori_loop(0, n_chunks, chunk_body, 0)


def kernel(x, table):
    b0, b1 = x.shape
    n_total = b0 * b1
    assert n_total % (NUM_WORKERS * CHUNK) == 0
    per_w = n_total // NUM_WORKERS
    n_chunks = per_w // CHUNK

    mesh = plsc.VectorSubcoreMesh(core_axis_name="c", subcore_axis_name="s")
    emb = functools.partial(
        pl.kernel,
        mesh=mesh,
        out_type=jax.ShapeDtypeStruct((n_total, D_MODEL), jnp.float32),
        scratch_types=[
            pltpu.VMEM((CHUNK,), jnp.int32),
            pltpu.VMEM((CHUNK, D_MODEL), jnp.float32),
            pltpu.SemaphoreType.DMA,
        ],
        compiler_params=pltpu.CompilerParams(use_tc_tiling_on_sc=False),
    )(functools.partial(_emb_body, per_w=per_w, n_chunks=n_chunks))

    out = emb(x.reshape(n_total), table)
    return out.reshape(b0, b1, D_MODEL)
